# baseline (speedup 1.0000x reference)
import jax
import jax.numpy as jnp
from jax.experimental import pallas as pl
from jax.experimental.pallas import tpu as pltpu

B = 8
L = 2048
D = 64
TOPK = 64
TM = 256
NT = L // TM
RS = L // 128
TR = TM // 128
NEG = float("-inf")


def _dot_t(a, w):
    return jax.lax.dot_general(a, w, (((1,), (1,)), ((), ())),
                               preferred_element_type=jnp.float32)


def _dot(a, b):
    return jax.lax.dot_general(a, b, (((1,), (0,)), ((), ())),
                               preferred_element_type=jnp.float32)


def _body(x_ref, qw_ref, qb_ref, kw_ref, kb_ref,
          p1w_ref, p1b_ref, p2w_ref, p2b_ref,
          x1w_ref, x1b_ref, x2w_ref, x2b_ref,
          r1w_ref, r1b_ref, r2w_ref, r2b_ref,
          out_ref,
          k_ref, mask_ref,
          best_ref, barg_ref, next_ref, narg_ref, nok_ref,
          vals_ref, rows_ref, cols_ref,
          xi_ref, xj_ref, smf_ref):
    scale = jnp.float32(0.125)
    ci = jax.lax.broadcasted_iota(jnp.int32, (TM, L), 1)

    def p1a(b, carry):
        xb = x_ref[pl.ds(b, 1)][0]
        maskf = jnp.sum(jnp.abs(xb), axis=1) != 0.0
        mask_ref[pl.ds(b, 1), :] = maskf.astype(jnp.int32)[None, :]
        k_ref[pl.ds(b, 1)] = (_dot_t(xb, kw_ref[...]) + kb_ref[...])[None]
        nok_ref[pl.ds(b, 1)] = jnp.ones((1, RS, 128), jnp.int32)
        return carry

    jax.lax.fori_loop(0, B, p1a, 0)

    def p1b(i, carry):
        b = i // NT
        ti = i - b * NT
        xt = x_ref[pl.ds(b, 1), pl.ds(ti * TM, TM), :][0]
        qt = _dot_t(xt, qw_ref[...]) + qb_ref[...]
        kb = k_ref[pl.ds(b, 1)][0]
        s = _dot_t(qt, kb) * scale
        rmask = (jnp.sum(jnp.abs(xt), axis=1) != 0.0)[:, None]
        cmask = mask_ref[pl.ds(b, 1), :] > 0
        s = jnp.where(rmask & cmask, s, NEG)
        m1 = jnp.max(s, axis=1)
        a1 = jnp.min(jnp.where(s == m1[:, None], ci, L), axis=1)
        s2 = jnp.where(ci == a1[:, None], NEG, s)
        m2 = jnp.max(s2, axis=1)
        a2 = jnp.min(jnp.where(s2 == m2[:, None], ci, L), axis=1)
        ix = (pl.ds(b, 1), pl.ds(ti * TR, TR))
        best_ref[ix] = m1.reshape(1, TR, 128)
        barg_ref[ix] = a1.astype(jnp.int32).reshape(1, TR, 128)
        next_ref[ix] = m2.reshape(1, TR, 128)
        narg_ref[ix] = a2.astype(jnp.int32).reshape(1, TR, 128)
        return carry

    jax.lax.fori_loop(0, B * NT, p1b, 0)

    flat3 = (jax.lax.broadcasted_iota(jnp.int32, (B, RS, 128), 1) * 128
             + jax.lax.broadcasted_iota(jnp.int32, (B, RS, 128), 2))
    b83 = jax.lax.broadcasted_iota(jnp.int32, (B, RS, 128), 0)
    b8c = jax.lax.broadcasted_iota(jnp.int32, (B, 1, 1), 0)
    it64 = jax.lax.broadcasted_iota(jnp.int32, (1, TOPK, 1), 1)
    ci_row = jax.lax.broadcasted_iota(jnp.int32, (1, L), 1)
    ciKL = jax.lax.broadcasted_iota(jnp.int32, (TOPK, L), 1)

    def _rmax(v):
        return jnp.max(jnp.max(v, axis=2, keepdims=True), axis=1,
                       keepdims=True)

    def _rmin(v):
        return jnp.min(jnp.min(v, axis=2, keepdims=True), axis=1,
                       keepdims=True)

    def _rsum(v):
        return jnp.sum(jnp.sum(v, axis=2, keepdims=True), axis=1,
                       keepdims=True)

    def step(t, carry):
        bestv = best_ref[...]
        m = _rmax(bestv)
        pos = jnp.where(bestv == m, flat3, L)
        r = _rmin(pos)
        sel = pos == r
        c = _rsum(jnp.where(sel, barg_ref[...], 0))
        vals_ref[...] = jnp.where(it64 == t, m, vals_ref[...])
        rows_ref[...] = jnp.where(it64 == t, r, rows_ref[...])
        cols_ref[...] = jnp.where(it64 == t, c, cols_ref[...])
        nr = _rsum(jnp.where(sel, 1 - nok_ref[...], 0))

        def promote():
            best_ref[...] = jnp.where(sel, next_ref[...], bestv)
            barg_ref[...] = jnp.where(sel, narg_ref[...], barg_ref[...])
            nok_ref[...] = jnp.where(sel, 0, nok_ref[...])

        def rescan():
            nbest = jnp.where(sel, next_ref[...], bestv)
            nbarg = jnp.where(sel, narg_ref[...], barg_ref[...])
            nnext = next_ref[...]
            nnarg = narg_ref[...]
            nnok = jnp.where(sel, 0, nok_ref[...])
            for b in range(B):
                need_b = jnp.sum(jnp.where(b8c == b, nr, 0)) > 0
                r_b = jnp.sum(jnp.where(b8c == b, r, 0))
                oh = (ci_row == r_b).astype(jnp.float32)
                q_sel = _dot_t(_dot(oh, x_ref[b]), qw_ref[...]) + qb_ref[...]
                s_row = _dot_t(q_sel, k_ref[b]) * scale
                cmask = mask_ref[b][None, :] > 0
                rl = rows_ref[b]
                cl = cols_ref[b]
                exc = jnp.max(jnp.where(
                    (cl == ciKL) & (rl == r_b), 1, 0),
                    axis=0)[None, :] > 0
                s_row = jnp.where(cmask & ~exc, s_row, NEG)
                m1 = jnp.max(s_row)
                a1 = jnp.min(jnp.where(s_row == m1, ci_row, L))
                s2 = jnp.where(ci_row == a1, NEG, s_row)
                m2 = jnp.max(s2)
                a2 = jnp.min(jnp.where(s2 == m2, ci_row, L))
                upd = sel & (b83 == b) & need_b
                nbest = jnp.where(upd, m1, nbest)
                nbarg = jnp.where(upd, a1.astype(jnp.int32), nbarg)
                nnext = jnp.where(upd, m2, nnext)
                nnarg = jnp.where(upd, a2.astype(jnp.int32), nnarg)
                nnok = jnp.where(upd, 1, nnok)
            best_ref[...] = nbest
            barg_ref[...] = nbarg
            next_ref[...] = nnext
            narg_ref[...] = nnarg
            nok_ref[...] = nnok

        jax.lax.cond(jnp.sum(nr) > 0, rescan, promote)
        return carry

    jax.lax.fori_loop(0, TOPK, step, 0)

    def p3(b, carry):
        rl = rows_ref[pl.ds(b, 1)][0]
        cl = cols_ref[pl.ds(b, 1)][0]
        xb = x_ref[pl.ds(b, 1)][0]
        oh_i = (rl == ciKL).astype(jnp.float32)
        oh_j = (cl == ciKL).astype(jnp.float32)
        xi_ref[pl.ds(b * TOPK, TOPK), :] = _dot(oh_i, xb)
        xj_ref[pl.ds(b * TOPK, TOPK), :] = _dot(oh_j, xb)
        smf_ref[pl.ds(b * TOPK, TOPK), :] = jnp.where(
            (rl - cl) == 0, 1.0, 0.0)
        return carry

    jax.lax.fori_loop(0, B, p3, 0)

    x_i = xi_ref[...]
    x_j = xj_ref[...]
    h_s = jax.nn.relu(_dot_t(x_i, p1w_ref[...]) + p1b_ref[...])
    f_s = _dot_t(h_s, p2w_ref[...]) + p2b_ref[...]
    x1w = x1w_ref[...]
    h_p = jax.nn.relu(_dot_t(x_i, x1w[:, :D]) + _dot_t(x_j, x1w[:, D:])
                      + x1b_ref[...])
    f_p = _dot_t(h_p, x2w_ref[...]) + x2b_ref[...]
    inter = jnp.where(smf_ref[...] > 0.5, f_s, f_p)

    vals = vals_ref[...]
    e = jnp.exp(vals - jnp.max(vals, axis=1, keepdims=True))
    w3 = e / jnp.sum(e, axis=1, keepdims=True)
    w = jnp.sum(w3, axis=2)

    wtile = jnp.concatenate([w] * B, axis=1)
    blk = (jax.lax.broadcasted_iota(jnp.int32, (B, B * TOPK), 1) // TOPK
           == jax.lax.broadcasted_iota(jnp.int32, (B, B * TOPK), 0))
    w_sel = jnp.where(blk, wtile, 0.0)
    pooled = _dot(w_sel, inter)

    o1 = jax.nn.relu(_dot_t(pooled, r1w_ref[...]) + r1b_ref[...])
    out_ref[...] = _dot_t(o1, r2w_ref[...]) + r2b_ref[...]


@jax.jit
def kernel(x, q_w, q_b, k_w, k_b, phi1_w, phi1_b, phi2_w, phi2_b,
           xi1_w, xi1_b, xi2_w, xi2_b, rho1_w, rho1_b, rho2_w, rho2_b):
    b2 = lambda v: v.reshape(1, -1)
    args = (x, q_w, b2(q_b), k_w, b2(k_b),
            phi1_w, b2(phi1_b), phi2_w, b2(phi2_b),
            xi1_w, b2(xi1_b), xi2_w, b2(xi2_b),
            rho1_w, b2(rho1_b), rho2_w, b2(rho2_b))
    return pl.pallas_call(
        _body,
        out_shape=jax.ShapeDtypeStruct((B, D), jnp.float32),
        scratch_shapes=[
            pltpu.VMEM((B, L, D), jnp.float32),
            pltpu.VMEM((B, L), jnp.int32),
            pltpu.VMEM((B, RS, 128), jnp.float32),
            pltpu.VMEM((B, RS, 128), jnp.int32),
            pltpu.VMEM((B, RS, 128), jnp.float32),
            pltpu.VMEM((B, RS, 128), jnp.int32),
            pltpu.VMEM((B, RS, 128), jnp.int32),
            pltpu.VMEM((B, TOPK, 1), jnp.float32),
            pltpu.VMEM((B, TOPK, 1), jnp.int32),
            pltpu.VMEM((B, TOPK, 1), jnp.int32),
            pltpu.VMEM((B * TOPK, D), jnp.float32),
            pltpu.VMEM((B * TOPK, D), jnp.float32),
            pltpu.VMEM((B * TOPK, 1), jnp.float32),
        ],
    )(*args)


# speedup vs baseline: 1.9696x; 1.9696x over previous
import jax
import jax.numpy as jnp
from jax.experimental import pallas as pl
from jax.experimental.pallas import tpu as pltpu

B = 8
L = 2048
D = 64
TOPK = 64
TM = 256
NT = L // TM
RS = L // 128
TR = TM // 128
NEG = float("-inf")
LL = L * L


def _dot_t(a, w):
    return jax.lax.dot_general(a, w, (((1,), (1,)), ((), ())),
                               preferred_element_type=jnp.float32)


def _dot(a, b):
    return jax.lax.dot_general(a, b, (((1,), (0,)), ((), ())),
                               preferred_element_type=jnp.float32)


def _body(x_ref, qw_ref, qb_ref, kw_ref, kb_ref,
          p1w_ref, p1b_ref, p2w_ref, p2b_ref,
          x1w_ref, x1b_ref, x2w_ref, x2b_ref,
          r1w_ref, r1b_ref, r2w_ref, r2b_ref,
          out_ref,
          k_ref, mask_ref, rmax_ref, rowsel_ref, block_ref,
          bestv_ref, bestc_ref,
          vals_ref, rows_ref, cols_ref,
          xi_ref, xj_ref, smf_ref):
    scale = jnp.float32(0.125)
    ciKL = jax.lax.broadcasted_iota(jnp.int32, (TOPK, L), 1)
    ci_row = jax.lax.broadcasted_iota(jnp.int32, (1, L), 1)

    def p1(b, carry):
        xb = x_ref[pl.ds(b, 1)][0]
        maskf = jnp.sum(jnp.abs(xb), axis=1) != 0.0
        mask_ref[pl.ds(b, 1), :] = maskf.astype(jnp.int32)[None, :]
        kb = _dot_t(xb, kw_ref[...]) + kb_ref[...]
        k_ref[pl.ds(b, 1)] = kb[None]
        for ti in range(NT):
            xt = x_ref[pl.ds(b, 1), ti * TM:(ti + 1) * TM, :][0]
            qt = _dot_t(xt, qw_ref[...]) + qb_ref[...]
            s = _dot_t(qt, kb) * scale
            rmask = maskf[ti * TM:(ti + 1) * TM][:, None]
            s = jnp.where(rmask & maskf[None, :], s, NEG)
            m1 = jnp.max(s, axis=1)
            rmax_ref[pl.ds(b, 1), ti * TR:(ti + 1) * TR, :] = (
                m1.reshape(1, TR, 128))
        return carry

    jax.lax.fori_loop(0, B, p1, 0)

    flat3 = (jax.lax.broadcasted_iota(jnp.int32, (B, RS, 128), 1) * 128
             + jax.lax.broadcasted_iota(jnp.int32, (B, RS, 128), 2))
    it64_3 = jax.lax.broadcasted_iota(jnp.int32, (1, TOPK, 1), 1)

    def p2a(t, carry):
        rm = rmax_ref[...]
        m = jnp.max(jnp.max(rm, axis=2, keepdims=True), axis=1,
                    keepdims=True)
        pos = jnp.where(rm == m, flat3, LL)
        r = jnp.min(jnp.min(pos, axis=2, keepdims=True), axis=1,
                    keepdims=True)
        sel = pos == r
        rowsel_ref[...] = jnp.where(it64_3 == t, r, rowsel_ref[...])
        rmax_ref[...] = jnp.where(sel, NEG, rm)
        return carry

    jax.lax.fori_loop(0, TOPK, p2a, 0)

    def p2b(b, carry):
        rsel = rowsel_ref[pl.ds(b, 1)][0]
        ohr = (rsel == ciKL).astype(jnp.float32)
        xb = x_ref[pl.ds(b, 1)][0]
        xsel = _dot(ohr, xb)
        qsel = _dot_t(xsel, qw_ref[...]) + qb_ref[...]
        s = _dot_t(qsel, k_ref[pl.ds(b, 1)][0]) * scale
        cmask = mask_ref[pl.ds(b, 1), :] > 0
        mcol = mask_ref[pl.ds(b, 1), :].astype(jnp.float32).reshape(L, 1)
        rmv = _dot(ohr, mcol) > 0.5
        s = jnp.where(rmv & cmask, s, NEG)
        block_ref[pl.ds(b, 1)] = s[None]
        m1 = jnp.max(s, axis=1)
        a1 = jnp.min(jnp.where(s == m1[:, None], ciKL, L), axis=1)
        bestv_ref[pl.ds(b, 1), :] = m1[None, :]
        bestc_ref[pl.ds(b, 1), :] = a1.astype(jnp.int32)[None, :]
        return carry

    jax.lax.fori_loop(0, B, p2b, 0)

    rowsel2 = jnp.sum(rowsel_ref[...], axis=2)
    it64_2 = jax.lax.broadcasted_iota(jnp.int32, (1, TOPK), 1)
    b8c = jax.lax.broadcasted_iota(jnp.int32, (B, 1), 0)
    b82 = jax.lax.broadcasted_iota(jnp.int32, (B, TOPK), 0)

    def p2c(t, carry):
        bestv = bestv_ref[...]
        bestc = bestc_ref[...]
        m = jnp.max(bestv, axis=1, keepdims=True)
        bflat = rowsel2 * L + bestc
        pos = jnp.where(bestv == m, bflat, LL)
        fmin = jnp.min(pos, axis=1, keepdims=True)
        sel = pos == fmin
        rblk = jnp.sum(jnp.where(sel, it64_2, 0), axis=1, keepdims=True)
        corig = jnp.sum(jnp.where(sel, bestc, 0), axis=1, keepdims=True)
        rorig = jnp.sum(jnp.where(sel, rowsel2, 0), axis=1, keepdims=True)
        vals_ref[...] = jnp.where(it64_2 == t, m, vals_ref[...])
        rows_ref[...] = jnp.where(it64_2 == t, rorig, rows_ref[...])
        cols_ref[...] = jnp.where(it64_2 == t, corig, cols_ref[...])
        for b in range(B):
            rb = jnp.sum(jnp.where(b8c == b, rblk, 0))
            cb = jnp.sum(jnp.where(b8c == b, corig, 0))
            row = block_ref[b, pl.ds(rb, 1), :]
            row = jnp.where(ci_row == cb, NEG, row)
            block_ref[b, pl.ds(rb, 1), :] = row
            m1 = jnp.max(row)
            a1 = jnp.min(jnp.where(row == m1, ci_row, L)).astype(jnp.int32)
            upd = sel & (b82 == b)
            bestv_ref[...] = jnp.where(upd, m1, bestv_ref[...])
            bestc_ref[...] = jnp.where(upd, a1, bestc_ref[...])
        return carry

    jax.lax.fori_loop(0, TOPK, p2c, 0)

    def p3(b, carry):
        rl = rows_ref[pl.ds(b, 1)].reshape(TOPK, 1)
        cl = cols_ref[pl.ds(b, 1)].reshape(TOPK, 1)
        xb = x_ref[pl.ds(b, 1)][0]
        oh_i = (rl == ciKL).astype(jnp.float32)
        oh_j = (cl == ciKL).astype(jnp.float32)
        xi_ref[pl.ds(b * TOPK, TOPK), :] = _dot(oh_i, xb)
        xj_ref[pl.ds(b * TOPK, TOPK), :] = _dot(oh_j, xb)
        smf_ref[pl.ds(b * TOPK, TOPK), :] = jnp.where(
            (rl - cl) == 0, 1.0, 0.0)
        return carry

    jax.lax.fori_loop(0, B, p3, 0)

    x_i = xi_ref[...]
    x_j = xj_ref[...]
    h_s = jax.nn.relu(_dot_t(x_i, p1w_ref[...]) + p1b_ref[...])
    f_s = _dot_t(h_s, p2w_ref[...]) + p2b_ref[...]
    x1w = x1w_ref[...]
    h_p = jax.nn.relu(_dot_t(x_i, x1w[:, :D]) + _dot_t(x_j, x1w[:, D:])
                      + x1b_ref[...])
    f_p = _dot_t(h_p, x2w_ref[...]) + x2b_ref[...]
    inter = jnp.where(smf_ref[...] > 0.5, f_s, f_p)

    vals = vals_ref[...]
    e = jnp.exp(vals - jnp.max(vals, axis=1, keepdims=True))
    w = e / jnp.sum(e, axis=1, keepdims=True)

    wtile = jnp.concatenate([w] * B, axis=1)
    blk = (jax.lax.broadcasted_iota(jnp.int32, (B, B * TOPK), 1) // TOPK
           == jax.lax.broadcasted_iota(jnp.int32, (B, B * TOPK), 0))
    w_sel = jnp.where(blk, wtile, 0.0)
    pooled = _dot(w_sel, inter)

    o1 = jax.nn.relu(_dot_t(pooled, r1w_ref[...]) + r1b_ref[...])
    out_ref[...] = _dot_t(o1, r2w_ref[...]) + r2b_ref[...]


@jax.jit
def kernel(x, q_w, q_b, k_w, k_b, phi1_w, phi1_b, phi2_w, phi2_b,
           xi1_w, xi1_b, xi2_w, xi2_b, rho1_w, rho1_b, rho2_w, rho2_b):
    b2 = lambda v: v.reshape(1, -1)
    args = (x, q_w, b2(q_b), k_w, b2(k_b),
            phi1_w, b2(phi1_b), phi2_w, b2(phi2_b),
            xi1_w, b2(xi1_b), xi2_w, b2(xi2_b),
            rho1_w, b2(rho1_b), rho2_w, b2(rho2_b))
    return pl.pallas_call(
        _body,
        out_shape=jax.ShapeDtypeStruct((B, D), jnp.float32),
        scratch_shapes=[
            pltpu.VMEM((B, L, D), jnp.float32),
            pltpu.VMEM((B, L), jnp.int32),
            pltpu.VMEM((B, RS, 128), jnp.float32),
            pltpu.VMEM((B, TOPK, 1), jnp.int32),
            pltpu.VMEM((B, TOPK, L), jnp.float32),
            pltpu.VMEM((B, TOPK), jnp.float32),
            pltpu.VMEM((B, TOPK), jnp.int32),
            pltpu.VMEM((B, TOPK), jnp.float32),
            pltpu.VMEM((B, TOPK), jnp.int32),
            pltpu.VMEM((B, TOPK), jnp.int32),
            pltpu.VMEM((B * TOPK, D), jnp.float32),
            pltpu.VMEM((B * TOPK, D), jnp.float32),
            pltpu.VMEM((B * TOPK, 1), jnp.float32),
        ],
    )(*args)


# speedup vs baseline: 2.0458x; 1.0387x over previous
import jax
import jax.numpy as jnp
from jax.experimental import pallas as pl
from jax.experimental.pallas import tpu as pltpu

B = 8
L = 2048
D = 64
TOPK = 64
TM = 256
NT = L // TM
RS = L // 128
TR = TM // 128
NEG = float("-inf")
LL = L * L


def _dot_t(a, w):
    return jax.lax.dot_general(a, w, (((1,), (1,)), ((), ())),
                               preferred_element_type=jnp.float32)


def _dot(a, b):
    return jax.lax.dot_general(a, b, (((1,), (0,)), ((), ())),
                               preferred_element_type=jnp.float32)


def _body(x_ref, qw_ref, qb_ref, kw_ref, kb_ref,
          p1w_ref, p1b_ref, p2w_ref, p2b_ref,
          x1w_ref, x1b_ref, x2w_ref, x2b_ref,
          r1w_ref, r1b_ref, r2w_ref, r2b_ref,
          out_ref,
          k_ref, mask_ref, rmax_ref, rowsel_ref, block_ref,
          bestv_ref, bestc_ref,
          vals_ref, rows_ref, cols_ref,
          xi_ref, xj_ref, smf_ref):
    scale = jnp.float32(0.125)
    ciKL = jax.lax.broadcasted_iota(jnp.int32, (TOPK, L), 1)
    ci_row = jax.lax.broadcasted_iota(jnp.int32, (1, L), 1)

    def p1(b, carry):
        xb = x_ref[pl.ds(b, 1)][0]
        maskf = jnp.sum(jnp.abs(xb), axis=1) != 0.0
        mask_ref[pl.ds(b, 1), :] = maskf.astype(jnp.int32)[None, :]
        kb = _dot_t(xb, kw_ref[...]) + kb_ref[...]
        k_ref[pl.ds(b, 1)] = kb[None]
        for ti in range(NT):
            xt = x_ref[pl.ds(b, 1), ti * TM:(ti + 1) * TM, :][0]
            qt = _dot_t(xt, qw_ref[...]) + qb_ref[...]
            s = _dot_t(qt, kb) * scale
            rmask = maskf[ti * TM:(ti + 1) * TM][:, None]
            s = jnp.where(rmask & maskf[None, :], s, NEG)
            m1 = jnp.max(s, axis=1)
            rmax_ref[pl.ds(b, 1), ti * TR:(ti + 1) * TR, :] = (
                m1.reshape(1, TR, 128))
        return carry

    jax.lax.fori_loop(0, B, p1, 0)

    flat3 = (jax.lax.broadcasted_iota(jnp.int32, (B, RS, 128), 1) * 128
             + jax.lax.broadcasted_iota(jnp.int32, (B, RS, 128), 2))
    it64_2 = jax.lax.broadcasted_iota(jnp.int32, (1, TOPK), 1)

    def p2a(t, carry):
        rm = rmax_ref[...]
        rsel = rowsel_ref[...]
        for k in range(2):
            m = jnp.max(jnp.max(rm, axis=2, keepdims=True), axis=1,
                        keepdims=True)
            pos = jnp.where(rm == m, flat3, LL)
            r = jnp.min(jnp.min(pos, axis=2, keepdims=True), axis=1,
                        keepdims=True)
            sel = pos == r
            r2 = jnp.sum(r, axis=2)
            rsel = jnp.where(it64_2 == 2 * t + k, r2, rsel)
            rm = jnp.where(sel, NEG, rm)
        rowsel_ref[...] = rsel
        rmax_ref[...] = rm
        return carry

    jax.lax.fori_loop(0, TOPK // 2, p2a, 0)

    def p2b(b, carry):
        rsel = rowsel_ref[pl.ds(b, 1)].reshape(TOPK, 1)
        ohr = (rsel == ciKL).astype(jnp.float32)
        xb = x_ref[pl.ds(b, 1)][0]
        xsel = _dot(ohr, xb)
        qsel = _dot_t(xsel, qw_ref[...]) + qb_ref[...]
        s = _dot_t(qsel, k_ref[pl.ds(b, 1)][0]) * scale
        cmask = mask_ref[pl.ds(b, 1), :] > 0
        mcol = mask_ref[pl.ds(b, 1), :].astype(jnp.float32).reshape(L, 1)
        rmv = _dot(ohr, mcol) > 0.5
        s = jnp.where(rmv & cmask, s, NEG)
        block_ref[pl.ds(b, 1)] = s[None]
        m1 = jnp.max(s, axis=1)
        a1 = jnp.min(jnp.where(s == m1[:, None], ciKL, L), axis=1)
        bestv_ref[pl.ds(b, 1), :] = m1[None, :]
        bestc_ref[pl.ds(b, 1), :] = a1.astype(jnp.int32)[None, :]
        return carry

    jax.lax.fori_loop(0, B, p2b, 0)

    rowsel2 = rowsel_ref[...]
    b8c = jax.lax.broadcasted_iota(jnp.int32, (B, 1), 0)
    b82 = jax.lax.broadcasted_iota(jnp.int32, (B, TOPK), 0)

    def p2c(t, carry):
        bestv = bestv_ref[...]
        bestc = bestc_ref[...]
        m = jnp.max(bestv, axis=1, keepdims=True)
        bflat = rowsel2 * L + bestc
        pos = jnp.where(bestv == m, bflat, LL)
        fmin = jnp.min(pos, axis=1, keepdims=True)
        sel = pos == fmin
        rblk = jnp.sum(jnp.where(sel, it64_2, 0), axis=1, keepdims=True)
        corig = jnp.sum(jnp.where(sel, bestc, 0), axis=1, keepdims=True)
        rorig = jnp.sum(jnp.where(sel, rowsel2, 0), axis=1, keepdims=True)
        vals_ref[...] = jnp.where(it64_2 == t, m, vals_ref[...])
        rows_ref[...] = jnp.where(it64_2 == t, rorig, rows_ref[...])
        cols_ref[...] = jnp.where(it64_2 == t, corig, cols_ref[...])
        for b in range(B):
            rb = jnp.sum(jnp.where(b8c == b, rblk, 0))
            cb = jnp.sum(jnp.where(b8c == b, corig, 0))
            row = block_ref[b, pl.ds(rb, 1), :]
            row = jnp.where(ci_row == cb, NEG, row)
            block_ref[b, pl.ds(rb, 1), :] = row
            m1 = jnp.max(row)
            a1 = jnp.min(jnp.where(row == m1, ci_row, L)).astype(jnp.int32)
            upd = sel & (b82 == b)
            bestv_ref[...] = jnp.where(upd, m1, bestv_ref[...])
            bestc_ref[...] = jnp.where(upd, a1, bestc_ref[...])
        return carry

    jax.lax.fori_loop(0, TOPK, p2c, 0)

    def p3(b, carry):
        rl = rows_ref[pl.ds(b, 1)].reshape(TOPK, 1)
        cl = cols_ref[pl.ds(b, 1)].reshape(TOPK, 1)
        xb = x_ref[pl.ds(b, 1)][0]
        oh_i = (rl == ciKL).astype(jnp.float32)
        oh_j = (cl == ciKL).astype(jnp.float32)
        xi_ref[pl.ds(b * TOPK, TOPK), :] = _dot(oh_i, xb)
        xj_ref[pl.ds(b * TOPK, TOPK), :] = _dot(oh_j, xb)
        smf_ref[pl.ds(b * TOPK, TOPK), :] = jnp.where(
            (rl - cl) == 0, 1.0, 0.0)
        return carry

    jax.lax.fori_loop(0, B, p3, 0)

    x_i = xi_ref[...]
    x_j = xj_ref[...]
    h_s = jax.nn.relu(_dot_t(x_i, p1w_ref[...]) + p1b_ref[...])
    f_s = _dot_t(h_s, p2w_ref[...]) + p2b_ref[...]
    x1w = x1w_ref[...]
    h_p = jax.nn.relu(_dot_t(x_i, x1w[:, :D]) + _dot_t(x_j, x1w[:, D:])
                      + x1b_ref[...])
    f_p = _dot_t(h_p, x2w_ref[...]) + x2b_ref[...]
    inter = jnp.where(smf_ref[...] > 0.5, f_s, f_p)

    vals = vals_ref[...]
    e = jnp.exp(vals - jnp.max(vals, axis=1, keepdims=True))
    w = e / jnp.sum(e, axis=1, keepdims=True)

    wtile = jnp.concatenate([w] * B, axis=1)
    blk = (jax.lax.broadcasted_iota(jnp.int32, (B, B * TOPK), 1) // TOPK
           == jax.lax.broadcasted_iota(jnp.int32, (B, B * TOPK), 0))
    w_sel = jnp.where(blk, wtile, 0.0)
    pooled = _dot(w_sel, inter)

    o1 = jax.nn.relu(_dot_t(pooled, r1w_ref[...]) + r1b_ref[...])
    out_ref[...] = _dot_t(o1, r2w_ref[...]) + r2b_ref[...]


@jax.jit
def kernel(x, q_w, q_b, k_w, k_b, phi1_w, phi1_b, phi2_w, phi2_b,
           xi1_w, xi1_b, xi2_w, xi2_b, rho1_w, rho1_b, rho2_w, rho2_b):
    b2 = lambda v: v.reshape(1, -1)
    args = (x, q_w, b2(q_b), k_w, b2(k_b),
            phi1_w, b2(phi1_b), phi2_w, b2(phi2_b),
            xi1_w, b2(xi1_b), xi2_w, b2(xi2_b),
            rho1_w, b2(rho1_b), rho2_w, b2(rho2_b))
    return pl.pallas_call(
        _body,
        out_shape=jax.ShapeDtypeStruct((B, D), jnp.float32),
        scratch_shapes=[
            pltpu.VMEM((B, L, D), jnp.float32),
            pltpu.VMEM((B, L), jnp.int32),
            pltpu.VMEM((B, RS, 128), jnp.float32),
            pltpu.VMEM((B, TOPK), jnp.int32),
            pltpu.VMEM((B, TOPK, L), jnp.float32),
            pltpu.VMEM((B, TOPK), jnp.float32),
            pltpu.VMEM((B, TOPK), jnp.int32),
            pltpu.VMEM((B, TOPK), jnp.float32),
            pltpu.VMEM((B, TOPK), jnp.int32),
            pltpu.VMEM((B, TOPK), jnp.int32),
            pltpu.VMEM((B * TOPK, D), jnp.float32),
            pltpu.VMEM((B * TOPK, D), jnp.float32),
            pltpu.VMEM((B * TOPK, 1), jnp.float32),
        ],
    )(*args)


# speedup vs baseline: 2.0677x; 1.0107x over previous
import jax
import jax.numpy as jnp
from jax.experimental import pallas as pl
from jax.experimental.pallas import tpu as pltpu

B = 8
L = 2048
D = 64
TOPK = 64
TM = 512
NT = L // TM
RS = L // 128
TR = TM // 128
NEG = float("-inf")
LL = L * L


def _dot_t(a, w):
    return jax.lax.dot_general(a, w, (((1,), (1,)), ((), ())),
                               preferred_element_type=jnp.float32)


def _dot(a, b):
    return jax.lax.dot_general(a, b, (((1,), (0,)), ((), ())),
                               preferred_element_type=jnp.float32)


def _body(x_ref, qw_ref, qb_ref, kw_ref, kb_ref,
          p1w_ref, p1b_ref, p2w_ref, p2b_ref,
          x1w_ref, x1b_ref, x2w_ref, x2b_ref,
          r1w_ref, r1b_ref, r2w_ref, r2b_ref,
          out_ref,
          k_ref, mask_ref, rmax_ref, rowsel_ref, block_ref,
          bestv_ref, bestc_ref,
          vals_ref, rows_ref, cols_ref,
          xi_ref, xj_ref, smf_ref):
    scale = jnp.float32(0.125)
    ciKL = jax.lax.broadcasted_iota(jnp.int32, (TOPK, L), 1)
    ci_row = jax.lax.broadcasted_iota(jnp.int32, (1, L), 1)

    def p1(b, carry):
        xb = x_ref[pl.ds(b, 1)][0]
        maskf = jnp.sum(jnp.abs(xb), axis=1) != 0.0
        mask_ref[pl.ds(b, 1), :] = maskf.astype(jnp.int32)[None, :]
        kb = _dot_t(xb, kw_ref[...]) + kb_ref[...]
        k_ref[pl.ds(b, 1)] = kb[None]
        for ti in range(NT):
            xt = x_ref[pl.ds(b, 1), ti * TM:(ti + 1) * TM, :][0]
            qt = _dot_t(xt, qw_ref[...]) + qb_ref[...]
            s = _dot_t(qt, kb) * scale
            rmask = maskf[ti * TM:(ti + 1) * TM][:, None]
            s = jnp.where(rmask & maskf[None, :], s, NEG)
            m1 = jnp.max(s, axis=1)
            rmax_ref[pl.ds(b, 1), ti * TR:(ti + 1) * TR, :] = (
                m1.reshape(1, TR, 128))
        return carry

    jax.lax.fori_loop(0, B, p1, 0)

    flat3 = (jax.lax.broadcasted_iota(jnp.int32, (B, RS, 128), 1) * 128
             + jax.lax.broadcasted_iota(jnp.int32, (B, RS, 128), 2))
    it64_2 = jax.lax.broadcasted_iota(jnp.int32, (1, TOPK), 1)

    def p2a(t, carry):
        rm = rmax_ref[...]
        rsel = rowsel_ref[...]
        for k in range(2):
            m = jnp.max(jnp.max(rm, axis=2, keepdims=True), axis=1,
                        keepdims=True)
            pos = jnp.where(rm == m, flat3, LL)
            r = jnp.min(jnp.min(pos, axis=2, keepdims=True), axis=1,
                        keepdims=True)
            sel = pos == r
            r2 = jnp.sum(r, axis=2)
            rsel = jnp.where(it64_2 == 2 * t + k, r2, rsel)
            rm = jnp.where(sel, NEG, rm)
        rowsel_ref[...] = rsel
        rmax_ref[...] = rm
        return carry

    jax.lax.fori_loop(0, TOPK // 2, p2a, 0)

    def p2b(b, carry):
        rsel = rowsel_ref[pl.ds(b, 1)].reshape(TOPK, 1)
        ohr = (rsel == ciKL).astype(jnp.float32)
        xb = x_ref[pl.ds(b, 1)][0]
        xsel = _dot(ohr, xb)
        qsel = _dot_t(xsel, qw_ref[...]) + qb_ref[...]
        s = _dot_t(qsel, k_ref[pl.ds(b, 1)][0]) * scale
        cmask = mask_ref[pl.ds(b, 1), :] > 0
        mcol = mask_ref[pl.ds(b, 1), :].astype(jnp.float32).reshape(L, 1)
        rmv = _dot(ohr, mcol) > 0.5
        s = jnp.where(rmv & cmask, s, NEG)
        block_ref[pl.ds(b, 1)] = s[None]
        m1 = jnp.max(s, axis=1)
        a1 = jnp.min(jnp.where(s == m1[:, None], ciKL, L), axis=1)
        bestv_ref[pl.ds(b, 1), :] = m1[None, :]
        bestc_ref[pl.ds(b, 1), :] = a1.astype(jnp.int32)[None, :]
        return carry

    jax.lax.fori_loop(0, B, p2b, 0)

    rowsel2 = rowsel_ref[...]
    b8c = jax.lax.broadcasted_iota(jnp.int32, (B, 1), 0)
    b82 = jax.lax.broadcasted_iota(jnp.int32, (B, TOPK), 0)

    def p2c(t, carry):
        bestv = bestv_ref[...]
        bestc = bestc_ref[...]
        m = jnp.max(bestv, axis=1, keepdims=True)
        bflat = rowsel2 * L + bestc
        pos = jnp.where(bestv == m, bflat, LL)
        fmin = jnp.min(pos, axis=1, keepdims=True)
        sel = pos == fmin
        rblk = jnp.sum(jnp.where(sel, it64_2, 0), axis=1, keepdims=True)
        corig = jnp.sum(jnp.where(sel, bestc, 0), axis=1, keepdims=True)
        rorig = jnp.sum(jnp.where(sel, rowsel2, 0), axis=1, keepdims=True)
        vals_ref[...] = jnp.where(it64_2 == t, m, vals_ref[...])
        rows_ref[...] = jnp.where(it64_2 == t, rorig, rows_ref[...])
        cols_ref[...] = jnp.where(it64_2 == t, corig, cols_ref[...])
        for b in range(B):
            rb = jnp.sum(jnp.where(b8c == b, rblk, 0))
            cb = jnp.sum(jnp.where(b8c == b, corig, 0))
            row = block_ref[b, pl.ds(rb, 1), :]
            row = jnp.where(ci_row == cb, NEG, row)
            block_ref[b, pl.ds(rb, 1), :] = row
            m1 = jnp.max(row)
            a1 = jnp.min(jnp.where(row == m1, ci_row, L)).astype(jnp.int32)
            upd = sel & (b82 == b)
            bestv_ref[...] = jnp.where(upd, m1, bestv_ref[...])
            bestc_ref[...] = jnp.where(upd, a1, bestc_ref[...])
        return carry

    jax.lax.fori_loop(0, TOPK, p2c, 0)

    def p3(b, carry):
        rl = rows_ref[pl.ds(b, 1)].reshape(TOPK, 1)
        cl = cols_ref[pl.ds(b, 1)].reshape(TOPK, 1)
        xb = x_ref[pl.ds(b, 1)][0]
        oh_i = (rl == ciKL).astype(jnp.float32)
        oh_j = (cl == ciKL).astype(jnp.float32)
        xi_ref[pl.ds(b * TOPK, TOPK), :] = _dot(oh_i, xb)
        xj_ref[pl.ds(b * TOPK, TOPK), :] = _dot(oh_j, xb)
        smf_ref[pl.ds(b * TOPK, TOPK), :] = jnp.where(
            (rl - cl) == 0, 1.0, 0.0)
        return carry

    jax.lax.fori_loop(0, B, p3, 0)

    x_i = xi_ref[...]
    x_j = xj_ref[...]
    h_s = jax.nn.relu(_dot_t(x_i, p1w_ref[...]) + p1b_ref[...])
    f_s = _dot_t(h_s, p2w_ref[...]) + p2b_ref[...]
    x1w = x1w_ref[...]
    h_p = jax.nn.relu(_dot_t(x_i, x1w[:, :D]) + _dot_t(x_j, x1w[:, D:])
                      + x1b_ref[...])
    f_p = _dot_t(h_p, x2w_ref[...]) + x2b_ref[...]
    inter = jnp.where(smf_ref[...] > 0.5, f_s, f_p)

    vals = vals_ref[...]
    e = jnp.exp(vals - jnp.max(vals, axis=1, keepdims=True))
    w = e / jnp.sum(e, axis=1, keepdims=True)

    wtile = jnp.concatenate([w] * B, axis=1)
    blk = (jax.lax.broadcasted_iota(jnp.int32, (B, B * TOPK), 1) // TOPK
           == jax.lax.broadcasted_iota(jnp.int32, (B, B * TOPK), 0))
    w_sel = jnp.where(blk, wtile, 0.0)
    pooled = _dot(w_sel, inter)

    o1 = jax.nn.relu(_dot_t(pooled, r1w_ref[...]) + r1b_ref[...])
    out_ref[...] = _dot_t(o1, r2w_ref[...]) + r2b_ref[...]


@jax.jit
def kernel(x, q_w, q_b, k_w, k_b, phi1_w, phi1_b, phi2_w, phi2_b,
           xi1_w, xi1_b, xi2_w, xi2_b, rho1_w, rho1_b, rho2_w, rho2_b):
    b2 = lambda v: v.reshape(1, -1)
    args = (x, q_w, b2(q_b), k_w, b2(k_b),
            phi1_w, b2(phi1_b), phi2_w, b2(phi2_b),
            xi1_w, b2(xi1_b), xi2_w, b2(xi2_b),
            rho1_w, b2(rho1_b), rho2_w, b2(rho2_b))
    return pl.pallas_call(
        _body,
        out_shape=jax.ShapeDtypeStruct((B, D), jnp.float32),
        scratch_shapes=[
            pltpu.VMEM((B, L, D), jnp.float32),
            pltpu.VMEM((B, L), jnp.int32),
            pltpu.VMEM((B, RS, 128), jnp.float32),
            pltpu.VMEM((B, TOPK), jnp.int32),
            pltpu.VMEM((B, TOPK, L), jnp.float32),
            pltpu.VMEM((B, TOPK), jnp.float32),
            pltpu.VMEM((B, TOPK), jnp.int32),
            pltpu.VMEM((B, TOPK), jnp.float32),
            pltpu.VMEM((B, TOPK), jnp.int32),
            pltpu.VMEM((B, TOPK), jnp.int32),
            pltpu.VMEM((B * TOPK, D), jnp.float32),
            pltpu.VMEM((B * TOPK, D), jnp.float32),
            pltpu.VMEM((B * TOPK, 1), jnp.float32),
        ],
    )(*args)
